# Initial kernel scaffold; baseline (speedup 1.0000x reference)
import jax
import jax.numpy as jnp
from jax.experimental import pallas as pl


def kernel(z, xyz, cg_z, cg_xyz, nbr_list, CG_nbr_list, mapping, num_CGs, emb_z, W_msg, W_rbf, mu_W1, mu_b1, mu_W2, mu_b2, sg_W1, sg_b1, sg_W2, sg_b2, emb_cgz, W_s, W_rbf2, W_gate):
    raise NotImplementedError("write your pallas kernel here")


# speedup vs baseline: 1.5260x; 1.5260x over previous
import functools

import jax
import jax.numpy as jnp
from jax import lax
from jax.experimental import pallas as pl
from jax.experimental.pallas import tpu as pltpu

F = 128
NRBF = 16
NCGP = 512
BE = 2528
EP = 128 * BE
BA = 2000
BC = 2000
F32 = jnp.float32


def _lane_iota(shape):
    return lax.broadcasted_iota(jnp.int32, shape, 1)


def _row_iota(shape):
    return lax.broadcasted_iota(jnp.int32, shape, 0)


def _dot(a, b):
    return jnp.dot(a, b, preferred_element_type=F32)


def _dot0(a, b):
    return lax.dot_general(a, b, (((0,), (0,)), ((), ())),
                           preferred_element_type=F32)


def _rbf_block(dist):
    centers = _lane_iota((1, NRBF)).astype(F32) * F32(1.8 / (NRBF - 1))
    return jnp.exp(-10.0 * (dist - centers) ** 2)


def _atoms_body(z_ref, m_ref, embp_ref, wmsg_ref, acc2_ref, cnt_ref, hp_ref):
    step = pl.program_id(0)

    @pl.when(step == 0)
    def _init():
        acc2_ref[...] = jnp.zeros_like(acc2_ref)
        cnt_ref[...] = jnp.zeros_like(cnt_ref)
        hp_ref[...] = jnp.tanh(_dot(embp_ref[...], wmsg_ref[...]))

    zc = z_ref[...]
    mc = m_ref[...]
    ohz = (zc == _lane_iota((BA, F))).astype(F32)
    s = _dot(ohz, embp_ref[...])
    ohm = (mc == _lane_iota((BA, NCGP))).astype(F32)
    acc2_ref[...] += _dot0(ohm, s)
    cnt_ref[...] += _dot0(ohm, jnp.ones((BA, F), F32))


def _atoms_call(z2, m2, embp, wmsg):
    n = z2.shape[0]
    grid = n // BA
    return pl.pallas_call(
        _atoms_body,
        grid=(grid,),
        in_specs=[
            pl.BlockSpec((BA, 1), lambda i: (i, 0)),
            pl.BlockSpec((BA, 1), lambda i: (i, 0)),
            pl.BlockSpec((F, F), lambda i: (0, 0)),
            pl.BlockSpec((F, F), lambda i: (0, 0)),
        ],
        out_specs=[
            pl.BlockSpec((NCGP, F), lambda i: (0, 0)),
            pl.BlockSpec((NCGP, F), lambda i: (0, 0)),
            pl.BlockSpec((F, F), lambda i: (0, 0)),
        ],
        out_shape=[
            jax.ShapeDtypeStruct((NCGP, F), F32),
            jax.ShapeDtypeStruct((NCGP, F), F32),
            jax.ShapeDtypeStruct((F, F), F32),
        ],
    )(z2, m2, embp, wmsg)


def _edges_body(rs_ref, rd_ref, hp_ref, wrbf_ref, acc_ref):
    step = pl.program_id(0)

    @pl.when(step == 0)
    def _init():
        acc_ref[...] = jnp.zeros_like(acc_ref)

    rs = rs_ref[...]
    rd = rd_ref[...]
    d = rd[:, 0:3] - rs[:, 0:3]
    dist = jnp.sqrt(jnp.sum(d * d, axis=1, keepdims=True) + 1e-12)
    rb = _rbf_block(dist)
    r = _dot(rb, wrbf_ref[...])
    zi = rd[:, 3:4].astype(jnp.int32)
    ohz = (zi == _lane_iota((BE, F))).astype(F32)
    hr = _dot(ohz, hp_ref[...])
    valid = (step * BE + _row_iota((BE, 1))) < 320000
    contrib = hr * r * valid.astype(F32)
    cg = rs[:, 3:4].astype(jnp.int32)
    ohc = (cg == _lane_iota((BE, NCGP))).astype(F32)
    acc_ref[...] += _dot0(ohc, contrib)


def _edges_call(rec_src, rec_dst, hp, wrbf):
    grid = EP // BE
    return pl.pallas_call(
        _edges_body,
        grid=(grid,),
        in_specs=[
            pl.BlockSpec((BE, 4), lambda i: (i, 0)),
            pl.BlockSpec((BE, 4), lambda i: (i, 0)),
            pl.BlockSpec((F, F), lambda i: (0, 0)),
            pl.BlockSpec((NRBF, F), lambda i: (0, 0)),
        ],
        out_specs=pl.BlockSpec((NCGP, F), lambda i: (0, 0)),
        out_shape=jax.ShapeDtypeStruct((NCGP, F), F32),
    )(rec_src, rec_dst, hp, wrbf)


def _fuse_body(acc_ref, acc2_ref, cnt_ref, mw1_ref, mb1_ref, mw2_ref, mb2_ref,
               sw1_ref, sb1_ref, sw2_ref, sb2_ref, ws_ref, embcg_ref, cgz_ref,
               mu_ref, sg_ref, cs_ref, first_ref):
    cnt = cnt_ref[...]
    s_i = (acc_ref[...] + acc2_ref[...]) / jnp.maximum(cnt, 1.0)
    t1 = jnp.tanh(_dot(s_i, mw1_ref[...]) + mb1_ref[...])
    mu_ref[...] = _dot(t1, mw2_ref[...]) + mb2_ref[...]
    t2 = jnp.tanh(_dot(s_i, sw1_ref[...]) + sb1_ref[...])
    logvar = _dot(t2, sw2_ref[...]) + sb2_ref[...]
    sg_ref[...] = 1e-12 + jnp.exp(logvar * 0.5)
    ohcg = (cgz_ref[...] == _lane_iota((NCGP, F))).astype(F32)
    cs_ref[...] = _dot(s_i, ws_ref[...]) + _dot(ohcg, embcg_ref[...])
    lt = (_row_iota((NCGP, NCGP)) > _lane_iota((NCGP, NCGP))).astype(F32)
    first_ref[...] = _dot(lt, cnt)


def _fuse_call(acc, acc2, cnt, mw1, mb1, mw2, mb2, sw1, sb1, sw2, sb2,
               ws, embcg, cgz2):
    full = lambda s: pl.BlockSpec(s, lambda: tuple(0 for _ in s))
    return pl.pallas_call(
        _fuse_body,
        in_specs=[full((NCGP, F)), full((NCGP, F)), full((NCGP, F)),
                  full((F, F)), full((1, F)), full((F, F)), full((1, F)),
                  full((F, F)), full((1, F)), full((F, F)), full((1, F)),
                  full((F, F)), full((F, F)), full((NCGP, 1))],
        out_specs=[full((NCGP, F))] * 4,
        out_shape=[jax.ShapeDtypeStruct((NCGP, F), F32)] * 4,
    )(acc, acc2, cnt, mw1, mb1, mw2, mb2, sw1, sb1, sw2, sb2, ws, embcg, cgz2)


def _cg_body(a_ref, b_ref, cs_ref, cxyz_ref, wrbf2_ref, wgate_ref, cnt_ref,
             first_ref, v0_ref, v1_ref, v2_ref, d4_ref):
    step = pl.program_id(0)

    @pl.when(step == 0)
    def _init():
        v0_ref[...] = jnp.zeros_like(v0_ref)
        v1_ref[...] = jnp.zeros_like(v1_ref)
        v2_ref[...] = jnp.zeros_like(v2_ref)

    ac = a_ref[...]
    bc = b_ref[...]
    oha = (ac == _lane_iota((BC, NCGP))).astype(F32)
    ohb = (bc == _lane_iota((BC, NCGP))).astype(F32)
    dv = _dot(ohb - oha, cxyz_ref[...])
    cdist = jnp.sqrt(jnp.sum(dv * dv, axis=1, keepdims=True) + 1e-12)
    crb = _rbf_block(cdist)
    csb = _dot(ohb, cs_ref[...])
    sm = csb * _dot(crb, wrbf2_ref[...])
    gate = jnp.tanh(_dot(sm, wgate_ref[...]))
    unit = dv / cdist
    v0_ref[...] += _dot0(oha, gate * unit[:, 0:1])
    v1_ref[...] += _dot0(oha, gate * unit[:, 1:2])
    v2_ref[...] += _dot0(oha, gate * unit[:, 2:3])

    @pl.when(step == pl.num_programs(0) - 1)
    def _fini():
        cnt = cnt_ref[...]
        lane = _lane_iota((NCGP, F))
        w = jnp.where(lane == F - 1,
                      jnp.maximum(cnt - (F - 1), 0.0),
                      (lane < cnt).astype(F32))
        inv = 1.0 / jnp.maximum(cnt[:, 0:1], 1.0)
        offs = [jnp.sum(v_ref[...] * w, axis=1, keepdims=True) * inv
                for v_ref in (v0_ref, v1_ref, v2_ref)]
        d4_ref[...] = jnp.concatenate(
            [cxyz_ref[:, 0:1] - offs[0],
             cxyz_ref[:, 1:2] - offs[1],
             cxyz_ref[:, 2:3] - offs[2],
             first_ref[:, 0:1]], axis=1)


def _cg_call(a2, b2, cs, cxyzp, wrbf2, wgate, cnt, firstb):
    ecg = a2.shape[0]
    grid = ecg // BC
    return pl.pallas_call(
        _cg_body,
        grid=(grid,),
        in_specs=[
            pl.BlockSpec((BC, 1), lambda i: (i, 0)),
            pl.BlockSpec((BC, 1), lambda i: (i, 0)),
            pl.BlockSpec((NCGP, F), lambda i: (0, 0)),
            pl.BlockSpec((NCGP, F), lambda i: (0, 0)),
            pl.BlockSpec((NRBF, F), lambda i: (0, 0)),
            pl.BlockSpec((F, F), lambda i: (0, 0)),
            pl.BlockSpec((NCGP, F), lambda i: (0, 0)),
            pl.BlockSpec((NCGP, F), lambda i: (0, 0)),
        ],
        out_specs=[pl.BlockSpec((NCGP, F), lambda i: (0, 0))] * 3
        + [pl.BlockSpec((NCGP, 4), lambda i: (0, 0))],
        out_shape=[jax.ShapeDtypeStruct((NCGP, F), F32)] * 3
        + [jax.ShapeDtypeStruct((NCGP, 4), F32)],
    )(a2, b2, cs, cxyzp, wrbf2, wgate, cnt, firstb)


def kernel(z, xyz, cg_z, cg_xyz, nbr_list, CG_nbr_list, mapping, num_CGs,
           emb_z, W_msg, W_rbf, mu_W1, mu_b1, mu_W2, mu_b2, sg_W1, sg_b1,
           sg_W2, sg_b2, emb_cgz, W_s, W_rbf2, W_gate):
    n = xyz.shape[0]
    e = nbr_list.shape[0]
    ecg = CG_nbr_list.shape[0]

    z = z.astype(jnp.int32)
    mapping = mapping.astype(jnp.int32)
    src = nbr_list[:, 0].astype(jnp.int32)
    dst = nbr_list[:, 1].astype(jnp.int32)
    src = jnp.pad(src, (0, EP - e))
    dst = jnp.pad(dst, (0, EP - e))
    t_src = jnp.concatenate([xyz, mapping.astype(F32)[:, None]], axis=1)
    t_dst = jnp.concatenate([xyz, z.astype(F32)[:, None]], axis=1)

    embp = jnp.zeros((F, F), F32).at[:100].set(emb_z)
    embcgp = jnp.zeros((F, F), F32).at[:100].set(emb_cgz)
    cxyzp = jnp.zeros((NCGP, F), F32).at[:500, :3].set(cg_xyz)
    z2 = z[:, None]
    m2 = mapping[:, None]
    cgz2 = jnp.pad(cg_z.astype(jnp.int32), (0, NCGP - 500))[:, None]
    a2 = CG_nbr_list[:, 0].astype(jnp.int32)[:, None]
    b2 = CG_nbr_list[:, 1].astype(jnp.int32)[:, None]

    rec_src = t_src[src]
    rec_dst = t_dst[dst]

    acc2, cnt, hp = _atoms_call(z2, m2, embp, W_msg)
    acc = _edges_call(rec_src, rec_dst, hp, W_rbf)
    mu, sigma, cs, firstb = _fuse_call(
        acc, acc2, cnt, mu_W1, mu_b1[None, :], mu_W2, mu_b2[None, :],
        sg_W1, sg_b1[None, :], sg_W2, sg_b2[None, :], W_s, embcgp, cgz2)
    v0, v1, v2, d4 = _cg_call(a2, b2, cs, cxyzp, W_rbf2, W_gate, cnt, firstb)

    first = d4[:, 3].astype(jnp.int32)
    chan = jnp.minimum(jnp.arange(n, dtype=jnp.int32) - first[mapping], F - 1)
    flat = mapping * F + chan
    cgv = jnp.stack([v0.reshape(-1), v1.reshape(-1), v2.reshape(-1)], axis=1)
    xyz_recon = cgv[flat] + d4[mapping, 0:3]

    return (mu[:500], sigma[:500], xyz, xyz_recon)
